# baseline (speedup 1.0000x reference)
import jax
import jax.numpy as jnp
from jax.experimental import pallas as pl
from jax.experimental.pallas import tpu as pltpu
from jax.experimental.pallas import tpu_sc as plsc

_WINDOW = 128


def _gather_kernel(num_indices: int, value_dim: int, dtype):
    mesh = plsc.VectorSubcoreMesh(core_axis_name="core", subcore_axis_name="subcore")

    @jax.jit
    def run(weight, indices):
        indices = indices.reshape((1, num_indices))

        @pl.kernel(
            out_type=jax.ShapeDtypeStruct((num_indices, value_dim), dtype),
            mesh=mesh,
            compiler_params=pltpu.CompilerParams(use_tc_tiling_on_sc=False),
        )
        def kernel(w_hbm, i_hbm, o_hbm):
            def body(i_vmem, o_vmem):
                pltpu.sync_copy(w_hbm.at[i_vmem.at[0]], o_vmem)

            pltpu.emit_pipeline(
                body,
                grid=(num_indices // _WINDOW,),
                in_specs=[
                    pl.BlockSpec((1, _WINDOW), index_map=lambda i: (0, i))
                ],
                out_specs=[
                    pl.BlockSpec((_WINDOW, value_dim), index_map=lambda i: (i, 0))
                ],
                core_axis_name=("core", "subcore"),
                dimension_semantics=(pltpu.PARALLEL,),
            )(i_hbm, o_hbm)

        return kernel(weight, indices)

    return run


def kernel(input_, weight):
    b, s = input_.shape
    num_indices = b * s
    value_dim = weight.shape[1]
    idx = input_.reshape((num_indices,)).astype(jnp.int32)
    out = _gather_kernel(num_indices, value_dim, weight.dtype)(weight, idx)
    return out.reshape((b, s, value_dim))


# speedup vs baseline: 1.0740x; 1.0740x over previous
import jax
import jax.numpy as jnp
from jax.experimental import pallas as pl
from jax.experimental.pallas import tpu as pltpu
from jax.experimental.pallas import tpu_sc as plsc

_WINDOW = 512


def _gather_kernel(num_indices: int, value_dim: int, dtype):
    mesh = plsc.VectorSubcoreMesh(core_axis_name="core", subcore_axis_name="subcore")

    @jax.jit
    def run(weight, indices):
        indices = indices.reshape((1, num_indices))

        @pl.kernel(
            out_type=jax.ShapeDtypeStruct((num_indices, value_dim), dtype),
            mesh=mesh,
            compiler_params=pltpu.CompilerParams(use_tc_tiling_on_sc=False),
        )
        def kernel(w_hbm, i_hbm, o_hbm):
            def body(i_vmem, o_vmem):
                pltpu.sync_copy(w_hbm.at[i_vmem.at[0]], o_vmem)

            pltpu.emit_pipeline(
                body,
                grid=(num_indices // _WINDOW,),
                in_specs=[
                    pl.BlockSpec((1, _WINDOW), index_map=lambda i: (0, i))
                ],
                out_specs=[
                    pl.BlockSpec((_WINDOW, value_dim), index_map=lambda i: (i, 0))
                ],
                core_axis_name=("core", "subcore"),
                dimension_semantics=(pltpu.PARALLEL,),
            )(i_hbm, o_hbm)

        return kernel(weight, indices)

    return run


def kernel(input_, weight):
    b, s = input_.shape
    num_indices = b * s
    value_dim = weight.shape[1]
    idx = input_.reshape((num_indices,)).astype(jnp.int32)
    out = _gather_kernel(num_indices, value_dim, weight.dtype)(weight, idx)
    return out.reshape((b, s, value_dim))
